# baseline (speedup 1.0000x reference)
import jax
import jax.numpy as jnp
from jax import lax
from jax.experimental import pallas as pl
from jax.experimental.pallas import tpu as pltpu
from jax.experimental.pallas import tpu_sc as plsc

N = 10000
E = 160000
H = 256
HALF = 128

CHUNK = 128
NS = 16
NC = 2
CHUNKS_PER_SUBCORE = 80
E_PAD = NS * CHUNKS_PER_SUBCORE * CHUNK
NROWS = E_PAD // CHUNK
NP = 10240
ROWS_PER_TILE = NP // NS

_BR = 1000
_GRID = N // _BR


def _mesh():
    return plsc.VectorSubcoreMesh(core_axis_name="c", subcore_axis_name="s")


def _make_agg():
    CPS = CHUNKS_PER_SUBCORE

    def body(hcat, src2, dst2, z128, out, acc, idxs, idxd, rows0, rows1,
             gs0, gs1, ss0, ss1):
        c = lax.axis_index("c")
        s = lax.axis_index("s")
        r0 = s * ROWS_PER_TILE
        src_bias = c * N

        pltpu.sync_copy(z128.at[pl.ds(r0, ROWS_PER_TILE)],
                        acc.at[pl.ds(r0, ROWS_PER_TILE)])
        plsc.subcore_barrier()

        def gstart(j, b):
            rb = rows0 if b == 0 else rows1
            sb = gs0 if b == 0 else gs1
            pltpu.async_copy(hcat.at[idxs.at[j]], rb, sb)

        def gwait(b):
            rb = rows0 if b == 0 else rows1
            sb = gs0 if b == 0 else gs1
            pltpu.make_async_copy(hcat.at[idxs.at[0]], rb, sb).wait()

        def sstart(j, b):
            rb = rows0 if b == 0 else rows1
            sb = ss0 if b == 0 else ss1
            pltpu.async_copy(rb, acc.at[idxd.at[j]], sb, add=True)

        def swait(b):
            rb = rows0 if b == 0 else rows1
            sb = ss0 if b == 0 else ss1
            pltpu.make_async_copy(rb, acc.at[idxd.at[0]], sb).wait()

        CPH = CPS // 2

        def phase(ph, carry):
            base = s * CPS + ph * CPH
            pltpu.sync_copy(src2.at[pl.ds(base, CPH)], idxs)
            pltpu.sync_copy(dst2.at[pl.ds(base, CPH)], idxd)

            def bias_step(j, cc):
                for k in range(CHUNK // 16):
                    sl = pl.ds(k * 16, 16)
                    idxs[j, sl] = idxs[j, sl] + src_bias
                return cc

            lax.fori_loop(0, CPH, bias_step, 0)

            gstart(0, 0)
            gstart(1, 1)
            gwait(0)
            sstart(0, 0)

            def step(j2, cc):
                jA = 2 * j2 + 1
                gwait(1)
                sstart(jA, 1)
                swait(0)
                gstart(jA + 1, 0)
                jB = jA + 1
                gwait(0)
                sstart(jB, 0)
                swait(1)
                gstart(jB + 1, 1)
                return cc

            lax.fori_loop(0, CPH // 2 - 1, step, 0)
            gwait(1)
            sstart(CPH - 1, 1)
            swait(0)
            swait(1)
            return carry

        lax.fori_loop(0, 2, phase, 0)
        plsc.subcore_barrier()

        o0 = c * NP + r0
        pltpu.sync_copy(acc.at[pl.ds(r0, ROWS_PER_TILE)],
                        out.at[pl.ds(o0, ROWS_PER_TILE)])

    return pl.kernel(
        body,
        out_type=[jax.ShapeDtypeStruct((2 * NP, HALF), jnp.float32)],
        mesh=_mesh(),
        scratch_types=[pltpu.VMEM_SHARED((NP, HALF), jnp.float32),
                       pltpu.VMEM((CPS // 2, CHUNK), jnp.int32),
                       pltpu.VMEM((CPS // 2, CHUNK), jnp.int32),
                       pltpu.VMEM((CHUNK, HALF), jnp.float32),
                       pltpu.VMEM((CHUNK, HALF), jnp.float32),
                       pltpu.SemaphoreType.DMA,
                       pltpu.SemaphoreType.DMA,
                       pltpu.SemaphoreType.DMA,
                       pltpu.SemaphoreType.DMA])


def _make_deg():
    half_rows = NROWS // 2
    cps = half_rows // NS

    def body(dst2, ones_h, z128, out, acc, idx_d, ones_v):
        c = lax.axis_index("c")
        s = lax.axis_index("s")
        r0 = s * ROWS_PER_TILE
        pltpu.sync_copy(z128.at[pl.ds(r0, ROWS_PER_TILE)],
                        acc.at[pl.ds(r0, ROWS_PER_TILE)])
        pltpu.sync_copy(ones_h, ones_v)
        plsc.subcore_barrier()

        def step(j, carry):
            row = c * half_rows + s * cps + j
            pltpu.sync_copy(dst2.at[row], idx_d)
            pltpu.sync_copy(ones_v, acc.at[idx_d], add=True)
            return carry

        lax.fori_loop(0, cps, step, 0)
        plsc.subcore_barrier()

        o0 = c * NP + r0
        pltpu.sync_copy(acc.at[pl.ds(r0, ROWS_PER_TILE)],
                        out.at[pl.ds(o0, ROWS_PER_TILE)])

    return pl.kernel(
        body,
        out_type=[jax.ShapeDtypeStruct((2 * NP, HALF), jnp.float32)],
        mesh=_mesh(),
        scratch_types=[pltpu.VMEM_SHARED((NP, HALF), jnp.float32),
                       pltpu.VMEM((CHUNK,), jnp.int32),
                       pltpu.VMEM((CHUNK, HALF), jnp.float32)])


def _full(shape):
    return pl.BlockSpec(shape, lambda i: (0, 0))


def _rows(width):
    return pl.BlockSpec((_BR, width), lambda i: (i, 0))


def _enc_body(s_ref, m_ref, ws, wm, b, h_ref, hs_ref):
    h = jnp.dot(s_ref[...], ws[...], preferred_element_type=jnp.float32)
    h += jnp.dot(m_ref[...], wm[...], preferred_element_type=jnp.float32)
    h = jnp.maximum(h + b[...], 0.0)
    h_ref[...] = h
    hs_ref[0] = h[:, :HALF]
    hs_ref[1] = h[:, HALF:]


def _z1_body(h0, sa, sb, dga, dgb, ws, wn, b, z_ref, s1_ref, s2_ref):
    i = pl.program_id(0)
    summ = jnp.concatenate([sa[...], sb[...]], axis=1)
    rdeg = 1.0 / jnp.maximum(dga[...] + dgb[...], 1.0)
    hn = summ * rdeg
    z = jnp.dot(h0[...], ws[...], preferred_element_type=jnp.float32)
    z += jnp.dot(hn, wn[...], preferred_element_type=jnp.float32)
    z += b[...]
    z_ref[...] = z
    bs1 = jnp.sum(z, axis=0, keepdims=True)
    bs2 = jnp.sum(z * z, axis=0, keepdims=True)

    @pl.when(i == 0)
    def _():
        s1_ref[...] = bs1
        s2_ref[...] = bs2

    @pl.when(i != 0)
    def _():
        s1_ref[...] += bs1
        s2_ref[...] += bs2


def _h1_body(z, s1, s2, g, bt, ws, wn, ps_ref, pn_ref):
    mu = s1[...] * (1.0 / N)
    var = s2[...] * (1.0 / N) - mu * mu
    sc = g[...] * lax.rsqrt(var + 1e-5)
    h1 = jnp.maximum((z[...] - mu) * sc + bt[...], 0.0)
    ps_ref[...] = jnp.dot(h1, ws[...], preferred_element_type=jnp.float32)
    pn = jnp.dot(h1, wn[...], preferred_element_type=jnp.float32)
    pn_ref[0] = pn[:, :HALF]
    pn_ref[1] = pn[:, HALF:]


def _z2_body(ps, sa, sb, dga, dgb, b, z_ref, s1_ref, s2_ref):
    i = pl.program_id(0)
    summ = jnp.concatenate([sa[...], sb[...]], axis=1)
    rdeg = 1.0 / jnp.maximum(dga[...] + dgb[...], 1.0)
    z = ps[...] + summ * rdeg + b[...]
    z_ref[...] = z
    bs1 = jnp.sum(z, axis=0, keepdims=True)
    bs2 = jnp.sum(z * z, axis=0, keepdims=True)

    @pl.when(i == 0)
    def _():
        s1_ref[...] = bs1
        s2_ref[...] = bs2

    @pl.when(i != 0)
    def _():
        s1_ref[...] += bs1
        s2_ref[...] += bs2


def _out_body(z, t1, t2, g, bt, h0, wra, wrb, br, wc1, bc1, wc2, bc2, o_ref):
    mu = t1[...] * (1.0 / N)
    var = t2[...] * (1.0 / N) - mu * mu
    h2 = jnp.maximum((z[...] - mu) * (g[...] * lax.rsqrt(var + 1e-5)) + bt[...],
                     0.0)
    hf = jnp.dot(h0[...], wra[...], preferred_element_type=jnp.float32)
    hf += jnp.dot(h2, wrb[...], preferred_element_type=jnp.float32)
    hf = jnp.maximum(hf + br[...], 0.0)
    hc = jnp.maximum(jnp.dot(hf, wc1[...], preferred_element_type=jnp.float32)
                     + bc1[...], 0.0)
    o_ref[...] = jnp.dot(hc, wc2[...], preferred_element_type=jnp.float32) \
        + bc2[...]


def _tc_call(body, in_specs, out_specs, out_shape):
    return pl.pallas_call(
        body,
        grid=(_GRID,),
        in_specs=in_specs,
        out_specs=out_specs,
        out_shape=out_shape,
        compiler_params=pltpu.CompilerParams(
            dimension_semantics=("arbitrary",)),
    )


def kernel(structural_features, multimodal_features, edge_index, W_in, b_in,
           W_self0, W_neigh0, b0, gamma0, beta0,
           W_self1, W_neigh1, b1, gamma1, beta1,
           W_rel, b_rel, W_c1, b_c1, W_c2, b_c2):
    f32 = jnp.float32
    src = edge_index[0]
    dst = edge_index[1]
    src2 = jnp.concatenate(
        [src, jnp.zeros((E_PAD - E,), jnp.int32)]).reshape(NROWS, CHUNK)
    dst2 = jnp.concatenate(
        [dst, jnp.full((E_PAD - E,), N, jnp.int32)]).reshape(NROWS, CHUNK)
    ones_h = jnp.ones((CHUNK, HALF), f32)
    z128 = jnp.zeros((NP, HALF), f32)

    WsT = W_in[:, :H].T
    WmT = W_in[:, H:].T
    Wself0T = W_self0.T
    Wneigh0T = W_neigh0.T
    Wself1T = W_self1.T
    Wneigh1T = W_neigh1.T
    WrAT = W_rel[:, :H].T
    WrBT = W_rel[:, H:].T
    Wc1T = W_c1.T
    Wc2T = W_c2.T
    b_in2 = b_in.reshape(1, H)
    b02 = b0.reshape(1, 2 * H)
    g02 = gamma0.reshape(1, 2 * H)
    be02 = beta0.reshape(1, 2 * H)
    b12 = b1.reshape(1, H)
    g12 = gamma1.reshape(1, H)
    be12 = beta1.reshape(1, H)
    brel2 = b_rel.reshape(1, H)
    bc12 = b_c1.reshape(1, H // 2)
    bc22 = b_c2.reshape(1, 64)

    h0, h0s = _tc_call(
        _enc_body,
        in_specs=[_rows(H), _rows(H), _full((H, H)), _full((H, H)),
                  _full((1, H))],
        out_specs=[_rows(H), pl.BlockSpec((2, _BR, HALF), lambda i: (0, i, 0))],
        out_shape=[jax.ShapeDtypeStruct((N, H), f32),
                   jax.ShapeDtypeStruct((2, N, HALF), f32)],
    )(structural_features, multimodal_features, WsT, WmT, b_in2)

    (degp,) = _make_deg()(dst2, ones_h, z128)
    dga = degp[:N, :1]
    dgb = degp[NP:NP + N, :1]

    (summ0,) = _make_agg()(h0s.reshape(2 * N, HALF), src2, dst2, z128)
    suma, sumb = summ0[:NP], summ0[NP:]

    Z1, s1, s2 = _tc_call(
        _z1_body,
        in_specs=[_rows(H), _rows(HALF), _rows(HALF), _rows(1), _rows(1),
                  _full((H, 2 * H)), _full((H, 2 * H)), _full((1, 2 * H))],
        out_specs=[_rows(2 * H), _full((1, 2 * H)), _full((1, 2 * H))],
        out_shape=[jax.ShapeDtypeStruct((N, 2 * H), f32),
                   jax.ShapeDtypeStruct((1, 2 * H), f32),
                   jax.ShapeDtypeStruct((1, 2 * H), f32)],
    )(h0, suma[:N], sumb[:N], dga, dgb, Wself0T, Wneigh0T, b02)

    P1self, p1ns = _tc_call(
        _h1_body,
        in_specs=[_rows(2 * H), _full((1, 2 * H)), _full((1, 2 * H)),
                  _full((1, 2 * H)), _full((1, 2 * H)),
                  _full((2 * H, H)), _full((2 * H, H))],
        out_specs=[_rows(H), pl.BlockSpec((2, _BR, HALF), lambda i: (0, i, 0))],
        out_shape=[jax.ShapeDtypeStruct((N, H), f32),
                   jax.ShapeDtypeStruct((2, N, HALF), f32)],
    )(Z1, s1, s2, g02, be02, Wself1T, Wneigh1T)

    (summ1,) = _make_agg()(p1ns.reshape(2 * N, HALF), src2, dst2, z128)
    s1a, s1b = summ1[:NP], summ1[NP:]

    Z2, t1, t2 = _tc_call(
        _z2_body,
        in_specs=[_rows(H), _rows(HALF), _rows(HALF), _rows(1), _rows(1),
                  _full((1, H))],
        out_specs=[_rows(H), _full((1, H)), _full((1, H))],
        out_shape=[jax.ShapeDtypeStruct((N, H), f32),
                   jax.ShapeDtypeStruct((1, H), f32),
                   jax.ShapeDtypeStruct((1, H), f32)],
    )(P1self, s1a[:N], s1b[:N], dga, dgb, b12)

    out = _tc_call(
        _out_body,
        in_specs=[_rows(H), _full((1, H)), _full((1, H)), _full((1, H)),
                  _full((1, H)), _rows(H), _full((H, H)), _full((H, H)),
                  _full((1, H)), _full((H, H // 2)), _full((1, H // 2)),
                  _full((H // 2, 64)), _full((1, 64))],
        out_specs=_rows(64),
        out_shape=jax.ShapeDtypeStruct((N, 64), f32),
    )(Z2, t1, t2, g12, be12, h0, WrAT, WrBT, brel2, Wc1T, bc12, Wc2T, bc22)

    return out


# speedup vs baseline: 1.0542x; 1.0542x over previous
import jax
import jax.numpy as jnp
from jax import lax
from jax.experimental import pallas as pl
from jax.experimental.pallas import tpu as pltpu
from jax.experimental.pallas import tpu_sc as plsc

N = 10000
E = 160000
H = 256
HALF = 128

CHUNK = 128
NS = 16
NC = 2
CHUNKS_PER_SUBCORE = 80
E_PAD = NS * CHUNKS_PER_SUBCORE * CHUNK
NROWS = E_PAD // CHUNK
NP = 10240
ROWS_PER_TILE = NP // NS

_BR = 1000
_GRID = N // _BR


def _mesh():
    return plsc.VectorSubcoreMesh(core_axis_name="c", subcore_axis_name="s")


def _make_agg():
    CPS = CHUNKS_PER_SUBCORE

    def body(hcat, src2, dst2, z128, out, acc, idxs, idxd, rows0, rows1,
             gs0, gs1, ss0, ss1):
        c = lax.axis_index("c")
        s = lax.axis_index("s")
        r0 = s * ROWS_PER_TILE
        src_bias = c * N

        pltpu.sync_copy(z128.at[pl.ds(r0, ROWS_PER_TILE)],
                        acc.at[pl.ds(r0, ROWS_PER_TILE)])
        plsc.subcore_barrier()

        def gstart(j, b):
            rb = rows0 if b == 0 else rows1
            sb = gs0 if b == 0 else gs1
            pltpu.async_copy(hcat.at[idxs.at[j]], rb, sb)

        def gwait(b):
            rb = rows0 if b == 0 else rows1
            sb = gs0 if b == 0 else gs1
            pltpu.make_async_copy(hcat.at[idxs.at[0]], rb, sb).wait()

        def sstart(j, b):
            rb = rows0 if b == 0 else rows1
            sb = ss0 if b == 0 else ss1
            pltpu.async_copy(rb, acc.at[idxd.at[j]], sb, add=True)

        def swait(b):
            rb = rows0 if b == 0 else rows1
            sb = ss0 if b == 0 else ss1
            pltpu.make_async_copy(rb, acc.at[idxd.at[0]], sb).wait()

        CPH = CPS // 2

        def phase(ph, carry):
            base = s * CPS + ph * CPH
            pltpu.sync_copy(src2.at[pl.ds(base, CPH)], idxs)
            pltpu.sync_copy(dst2.at[pl.ds(base, CPH)], idxd)

            def bias_step(j, cc):
                for k in range(CHUNK // 16):
                    sl = pl.ds(k * 16, 16)
                    idxs[j, sl] = idxs[j, sl] + src_bias
                return cc

            lax.fori_loop(0, CPH, bias_step, 0)

            gstart(0, 0)
            gstart(1, 1)
            gwait(0)
            sstart(0, 0)

            def step(j2, cc):
                jA = 2 * j2 + 1
                gwait(1)
                sstart(jA, 1)
                swait(0)
                gstart(jA + 1, 0)
                jB = jA + 1
                gwait(0)
                sstart(jB, 0)
                swait(1)
                gstart(jB + 1, 1)
                return cc

            lax.fori_loop(0, CPH // 2 - 1, step, 0)
            gwait(1)
            sstart(CPH - 1, 1)
            swait(0)
            swait(1)
            return carry

        lax.fori_loop(0, 2, phase, 0)
        plsc.subcore_barrier()

        o0 = c * NP + r0
        pltpu.sync_copy(acc.at[pl.ds(r0, ROWS_PER_TILE)],
                        out.at[pl.ds(o0, ROWS_PER_TILE)])

    return pl.kernel(
        body,
        out_type=[jax.ShapeDtypeStruct((2 * NP, HALF), jnp.float32)],
        mesh=_mesh(),
        scratch_types=[pltpu.VMEM_SHARED((NP, HALF), jnp.float32),
                       pltpu.VMEM((CPS // 2, CHUNK), jnp.int32),
                       pltpu.VMEM((CPS // 2, CHUNK), jnp.int32),
                       pltpu.VMEM((CHUNK, HALF), jnp.float32),
                       pltpu.VMEM((CHUNK, HALF), jnp.float32),
                       pltpu.SemaphoreType.DMA,
                       pltpu.SemaphoreType.DMA,
                       pltpu.SemaphoreType.DMA,
                       pltpu.SemaphoreType.DMA])


def _make_deg():
    half_rows = NROWS // 2
    cps = half_rows // NS

    def body(dst2, ones_h, z128, out, acc, idx_d, ones_v):
        c = lax.axis_index("c")
        s = lax.axis_index("s")
        r0 = s * ROWS_PER_TILE
        pltpu.sync_copy(z128.at[pl.ds(r0, ROWS_PER_TILE)],
                        acc.at[pl.ds(r0, ROWS_PER_TILE)])
        pltpu.sync_copy(ones_h, ones_v)
        plsc.subcore_barrier()

        def step(j, carry):
            row = c * half_rows + s * cps + j
            pltpu.sync_copy(dst2.at[row], idx_d)
            pltpu.sync_copy(ones_v, acc.at[idx_d], add=True)
            return carry

        lax.fori_loop(0, cps, step, 0)
        plsc.subcore_barrier()

        o0 = c * NP + r0
        pltpu.sync_copy(acc.at[pl.ds(r0, ROWS_PER_TILE)],
                        out.at[pl.ds(o0, ROWS_PER_TILE)])

    return pl.kernel(
        body,
        out_type=[jax.ShapeDtypeStruct((2 * NP, HALF), jnp.float32)],
        mesh=_mesh(),
        scratch_types=[pltpu.VMEM_SHARED((NP, HALF), jnp.float32),
                       pltpu.VMEM((CHUNK,), jnp.int32),
                       pltpu.VMEM((CHUNK, HALF), jnp.float32)])


def _full(shape):
    return pl.BlockSpec(shape, lambda i: (0, 0))


def _rows(width):
    return pl.BlockSpec((_BR, width), lambda i: (i, 0))


def _enc_body(s_ref, m_ref, ws, wm, b, h_ref, hs_ref):
    bf = jnp.bfloat16
    h = jnp.dot(s_ref[...].astype(bf), ws[...], preferred_element_type=jnp.float32)
    h += jnp.dot(m_ref[...].astype(bf), wm[...], preferred_element_type=jnp.float32)
    h = jnp.maximum(h + b[...], 0.0)
    h_ref[...] = h
    hs_ref[0] = h[:, :HALF]
    hs_ref[1] = h[:, HALF:]


def _z1_body(h0, sa, sb, dga, dgb, ws, wn, b, z_ref, s1_ref, s2_ref):
    i = pl.program_id(0)
    summ = jnp.concatenate([sa[...], sb[...]], axis=1)
    rdeg = 1.0 / jnp.maximum(dga[...] + dgb[...], 1.0)
    hn = summ * rdeg
    bf = jnp.bfloat16
    z = jnp.dot(h0[...].astype(bf), ws[...], preferred_element_type=jnp.float32)
    z += jnp.dot(hn.astype(bf), wn[...], preferred_element_type=jnp.float32)
    z += b[...]
    z_ref[...] = z
    bs1 = jnp.sum(z, axis=0, keepdims=True)
    bs2 = jnp.sum(z * z, axis=0, keepdims=True)

    @pl.when(i == 0)
    def _():
        s1_ref[...] = bs1
        s2_ref[...] = bs2

    @pl.when(i != 0)
    def _():
        s1_ref[...] += bs1
        s2_ref[...] += bs2


def _h1_body(z, s1, s2, g, bt, ws, wn, ps_ref, pn_ref):
    mu = s1[...] * (1.0 / N)
    var = s2[...] * (1.0 / N) - mu * mu
    sc = g[...] * lax.rsqrt(var + 1e-5)
    h1 = jnp.maximum((z[...] - mu) * sc + bt[...], 0.0)
    h1b = h1.astype(jnp.bfloat16)
    ps_ref[...] = jnp.dot(h1b, ws[...], preferred_element_type=jnp.float32)
    pn = jnp.dot(h1b, wn[...], preferred_element_type=jnp.float32)
    pn_ref[0] = pn[:, :HALF]
    pn_ref[1] = pn[:, HALF:]


def _z2_body(ps, sa, sb, dga, dgb, b, z_ref, s1_ref, s2_ref):
    i = pl.program_id(0)
    summ = jnp.concatenate([sa[...], sb[...]], axis=1)
    rdeg = 1.0 / jnp.maximum(dga[...] + dgb[...], 1.0)
    z = ps[...] + summ * rdeg + b[...]
    z_ref[...] = z
    bs1 = jnp.sum(z, axis=0, keepdims=True)
    bs2 = jnp.sum(z * z, axis=0, keepdims=True)

    @pl.when(i == 0)
    def _():
        s1_ref[...] = bs1
        s2_ref[...] = bs2

    @pl.when(i != 0)
    def _():
        s1_ref[...] += bs1
        s2_ref[...] += bs2


def _out_body(z, t1, t2, g, bt, h0, wra, wrb, br, wc1, bc1, wc2, bc2, o_ref):
    mu = t1[...] * (1.0 / N)
    var = t2[...] * (1.0 / N) - mu * mu
    h2 = jnp.maximum((z[...] - mu) * (g[...] * lax.rsqrt(var + 1e-5)) + bt[...],
                     0.0)
    bf = jnp.bfloat16
    hf = jnp.dot(h0[...].astype(bf), wra[...], preferred_element_type=jnp.float32)
    hf += jnp.dot(h2.astype(bf), wrb[...], preferred_element_type=jnp.float32)
    hf = jnp.maximum(hf + br[...], 0.0)
    hc = jnp.maximum(jnp.dot(hf.astype(bf), wc1[...],
                             preferred_element_type=jnp.float32)
                     + bc1[...], 0.0)
    o_ref[...] = jnp.dot(hc.astype(bf), wc2[...],
                         preferred_element_type=jnp.float32) + bc2[...]


def _tc_call(body, in_specs, out_specs, out_shape):
    return pl.pallas_call(
        body,
        grid=(_GRID,),
        in_specs=in_specs,
        out_specs=out_specs,
        out_shape=out_shape,
        compiler_params=pltpu.CompilerParams(
            dimension_semantics=("arbitrary",)),
    )


def kernel(structural_features, multimodal_features, edge_index, W_in, b_in,
           W_self0, W_neigh0, b0, gamma0, beta0,
           W_self1, W_neigh1, b1, gamma1, beta1,
           W_rel, b_rel, W_c1, b_c1, W_c2, b_c2):
    f32 = jnp.float32
    src = edge_index[0]
    dst = edge_index[1]
    src2 = jnp.concatenate(
        [src, jnp.zeros((E_PAD - E,), jnp.int32)]).reshape(NROWS, CHUNK)
    dst2 = jnp.concatenate(
        [dst, jnp.full((E_PAD - E,), N, jnp.int32)]).reshape(NROWS, CHUNK)
    ones_h = jnp.ones((CHUNK, HALF), f32)
    z128 = jnp.zeros((NP, HALF), f32)

    bf = jnp.bfloat16
    WsT = W_in[:, :H].T.astype(bf)
    WmT = W_in[:, H:].T.astype(bf)
    Wself0T = W_self0.T.astype(bf)
    Wneigh0T = W_neigh0.T.astype(bf)
    Wself1T = W_self1.T.astype(bf)
    Wneigh1T = W_neigh1.T.astype(bf)
    WrAT = W_rel[:, :H].T.astype(bf)
    WrBT = W_rel[:, H:].T.astype(bf)
    Wc1T = W_c1.T.astype(bf)
    Wc2T = W_c2.T.astype(bf)
    b_in2 = b_in.reshape(1, H)
    b02 = b0.reshape(1, 2 * H)
    g02 = gamma0.reshape(1, 2 * H)
    be02 = beta0.reshape(1, 2 * H)
    b12 = b1.reshape(1, H)
    g12 = gamma1.reshape(1, H)
    be12 = beta1.reshape(1, H)
    brel2 = b_rel.reshape(1, H)
    bc12 = b_c1.reshape(1, H // 2)
    bc22 = b_c2.reshape(1, 64)

    h0, h0s = _tc_call(
        _enc_body,
        in_specs=[_rows(H), _rows(H), _full((H, H)), _full((H, H)),
                  _full((1, H))],
        out_specs=[_rows(H), pl.BlockSpec((2, _BR, HALF), lambda i: (0, i, 0))],
        out_shape=[jax.ShapeDtypeStruct((N, H), f32),
                   jax.ShapeDtypeStruct((2, N, HALF), f32)],
    )(structural_features, multimodal_features, WsT, WmT, b_in2)

    (degp,) = _make_deg()(dst2, ones_h, z128)
    dga = degp[:N, :1]
    dgb = degp[NP:NP + N, :1]

    (summ0,) = _make_agg()(h0s.reshape(2 * N, HALF), src2, dst2, z128)
    suma, sumb = summ0[:NP], summ0[NP:]

    Z1, s1, s2 = _tc_call(
        _z1_body,
        in_specs=[_rows(H), _rows(HALF), _rows(HALF), _rows(1), _rows(1),
                  _full((H, 2 * H)), _full((H, 2 * H)), _full((1, 2 * H))],
        out_specs=[_rows(2 * H), _full((1, 2 * H)), _full((1, 2 * H))],
        out_shape=[jax.ShapeDtypeStruct((N, 2 * H), f32),
                   jax.ShapeDtypeStruct((1, 2 * H), f32),
                   jax.ShapeDtypeStruct((1, 2 * H), f32)],
    )(h0, suma[:N], sumb[:N], dga, dgb, Wself0T, Wneigh0T, b02)

    P1self, p1ns = _tc_call(
        _h1_body,
        in_specs=[_rows(2 * H), _full((1, 2 * H)), _full((1, 2 * H)),
                  _full((1, 2 * H)), _full((1, 2 * H)),
                  _full((2 * H, H)), _full((2 * H, H))],
        out_specs=[_rows(H), pl.BlockSpec((2, _BR, HALF), lambda i: (0, i, 0))],
        out_shape=[jax.ShapeDtypeStruct((N, H), f32),
                   jax.ShapeDtypeStruct((2, N, HALF), f32)],
    )(Z1, s1, s2, g02, be02, Wself1T, Wneigh1T)

    (summ1,) = _make_agg()(p1ns.reshape(2 * N, HALF), src2, dst2, z128)
    s1a, s1b = summ1[:NP], summ1[NP:]

    Z2, t1, t2 = _tc_call(
        _z2_body,
        in_specs=[_rows(H), _rows(HALF), _rows(HALF), _rows(1), _rows(1),
                  _full((1, H))],
        out_specs=[_rows(H), _full((1, H)), _full((1, H))],
        out_shape=[jax.ShapeDtypeStruct((N, H), f32),
                   jax.ShapeDtypeStruct((1, H), f32),
                   jax.ShapeDtypeStruct((1, H), f32)],
    )(P1self, s1a[:N], s1b[:N], dga, dgb, b12)

    out = _tc_call(
        _out_body,
        in_specs=[_rows(H), _full((1, H)), _full((1, H)), _full((1, H)),
                  _full((1, H)), _rows(H), _full((H, H)), _full((H, H)),
                  _full((1, H)), _full((H, H // 2)), _full((1, H // 2)),
                  _full((H // 2, 64)), _full((1, 64))],
        out_specs=_rows(64),
        out_shape=jax.ShapeDtypeStruct((N, 64), f32),
    )(Z2, t1, t2, g12, be12, h0, WrAT, WrBT, brel2, Wc1T, bc12, Wc2T, bc22)

    return out


# speedup vs baseline: 1.0838x; 1.0281x over previous
import jax
import jax.numpy as jnp
from jax import lax
from jax.experimental import pallas as pl
from jax.experimental.pallas import tpu as pltpu
from jax.experimental.pallas import tpu_sc as plsc

N = 10000
E = 160000
H = 256
HALF = 128

CHUNK = 128
CH_A = 64
NS = 16
NC = 2
E_PAD = 163840
NROWS = E_PAD // CHUNK
NROWS_A = E_PAD // CH_A
CPS_A = NROWS_A // NS
NP = 10240
ROWS_PER_TILE = NP // NS

_BR = 1000
_GRID = N // _BR


def _mesh():
    return plsc.VectorSubcoreMesh(core_axis_name="c", subcore_axis_name="s")


def _make_agg():
    CPH = 40
    NPH = CPS_A // CPH

    def body(hcat, src2, dst2, z128, out, acc, idxs, idxd,
             rows0, rows1, rows2, rows3, gs0, gs1, gs2, gs3,
             ss0, ss1, ss2, ss3):
        c = lax.axis_index("c")
        s = lax.axis_index("s")
        r0 = s * ROWS_PER_TILE
        src_bias = c * N
        rows = (rows0, rows1, rows2, rows3)
        gsem = (gs0, gs1, gs2, gs3)
        ssem = (ss0, ss1, ss2, ss3)

        pltpu.sync_copy(z128.at[pl.ds(r0, ROWS_PER_TILE)],
                        acc.at[pl.ds(r0, ROWS_PER_TILE)])
        plsc.subcore_barrier()

        def gstart(j, b):
            pltpu.async_copy(hcat.at[idxs.at[j]], rows[b], gsem[b])

        def gwait(b):
            pltpu.make_async_copy(hcat.at[idxs.at[0]], rows[b],
                                  gsem[b]).wait()

        def sstart(j, b):
            pltpu.async_copy(rows[b], acc.at[idxd.at[j]], ssem[b], add=True)

        def swait(b):
            pltpu.make_async_copy(rows[b], acc.at[idxd.at[0]],
                                  ssem[b]).wait()

        def phase(ph, carry):
            base = s * CPS_A + ph * CPH
            pltpu.sync_copy(src2.at[pl.ds(base, CPH)], idxs)
            pltpu.sync_copy(dst2.at[pl.ds(base, CPH)], idxd)

            def bias_step(j, cc):
                for k in range(CH_A // 16):
                    sl = pl.ds(k * 16, 16)
                    idxs[j, sl] = idxs[j, sl] + src_bias
                return cc

            lax.fori_loop(0, CPH, bias_step, 0)

            gstart(0, 0)
            gstart(1, 1)
            gstart(2, 2)
            gwait(0)
            sstart(0, 0)
            gstart(3, 3)

            def step4(j2, cc):
                for t in range(4):
                    j = 1 + 4 * j2 + t
                    b = (1 + t) % 4
                    bn = (b + 3) % 4
                    gwait(b)
                    swait(bn)
                    gstart(j + 3, bn)
                    sstart(j, b)
                return cc

            lax.fori_loop(0, (CPH - 4) // 4, step4, 0)
            for j, b in ((CPH - 3, 1), (CPH - 2, 2), (CPH - 1, 3)):
                gwait(b)
                sstart(j, b)
            for b in range(4):
                swait(b)
            return carry

        lax.fori_loop(0, NPH, phase, 0)
        plsc.subcore_barrier()

        o0 = c * NP + r0
        pltpu.sync_copy(acc.at[pl.ds(r0, ROWS_PER_TILE)],
                        out.at[pl.ds(o0, ROWS_PER_TILE)])

    return pl.kernel(
        body,
        out_type=[jax.ShapeDtypeStruct((2 * NP, HALF), jnp.float32)],
        mesh=_mesh(),
        scratch_types=[pltpu.VMEM_SHARED((NP, HALF), jnp.float32),
                       pltpu.VMEM((CPH, CH_A), jnp.int32),
                       pltpu.VMEM((CPH, CH_A), jnp.int32)]
        + [pltpu.VMEM((CH_A, HALF), jnp.float32)] * 4
        + [pltpu.SemaphoreType.DMA] * 8)


def _make_deg():
    half_rows = NROWS // 2
    cps = half_rows // NS

    def body(dst2, ones_h, z128, out, acc, idx_d, ones_v):
        c = lax.axis_index("c")
        s = lax.axis_index("s")
        r0 = s * ROWS_PER_TILE
        pltpu.sync_copy(z128.at[pl.ds(r0, ROWS_PER_TILE)],
                        acc.at[pl.ds(r0, ROWS_PER_TILE)])
        pltpu.sync_copy(ones_h, ones_v)
        plsc.subcore_barrier()

        def step(j, carry):
            row = c * half_rows + s * cps + j
            pltpu.sync_copy(dst2.at[row], idx_d)
            pltpu.sync_copy(ones_v, acc.at[idx_d], add=True)
            return carry

        lax.fori_loop(0, cps, step, 0)
        plsc.subcore_barrier()

        o0 = c * NP + r0
        pltpu.sync_copy(acc.at[pl.ds(r0, ROWS_PER_TILE)],
                        out.at[pl.ds(o0, ROWS_PER_TILE)])

    return pl.kernel(
        body,
        out_type=[jax.ShapeDtypeStruct((2 * NP, HALF), jnp.float32)],
        mesh=_mesh(),
        scratch_types=[pltpu.VMEM_SHARED((NP, HALF), jnp.float32),
                       pltpu.VMEM((CHUNK,), jnp.int32),
                       pltpu.VMEM((CHUNK, HALF), jnp.float32)])


def _full(shape):
    return pl.BlockSpec(shape, lambda i: (0, 0))


def _rows(width):
    return pl.BlockSpec((_BR, width), lambda i: (i, 0))


def _enc_body(s_ref, m_ref, ws, wm, b, h_ref, hs_ref):
    bf = jnp.bfloat16
    h = jnp.dot(s_ref[...].astype(bf), ws[...], preferred_element_type=jnp.float32)
    h += jnp.dot(m_ref[...].astype(bf), wm[...], preferred_element_type=jnp.float32)
    h = jnp.maximum(h + b[...], 0.0)
    h_ref[...] = h
    hs_ref[0] = h[:, :HALF]
    hs_ref[1] = h[:, HALF:]


def _z1_body(h0, sa, sb, dga, dgb, ws, wn, b, z_ref, s1_ref, s2_ref):
    i = pl.program_id(0)
    summ = jnp.concatenate([sa[...], sb[...]], axis=1)
    rdeg = 1.0 / jnp.maximum(dga[...] + dgb[...], 1.0)
    hn = summ * rdeg
    bf = jnp.bfloat16
    z = jnp.dot(h0[...].astype(bf), ws[...], preferred_element_type=jnp.float32)
    z += jnp.dot(hn.astype(bf), wn[...], preferred_element_type=jnp.float32)
    z += b[...]
    z_ref[...] = z
    bs1 = jnp.sum(z, axis=0, keepdims=True)
    bs2 = jnp.sum(z * z, axis=0, keepdims=True)

    @pl.when(i == 0)
    def _():
        s1_ref[...] = bs1
        s2_ref[...] = bs2

    @pl.when(i != 0)
    def _():
        s1_ref[...] += bs1
        s2_ref[...] += bs2


def _h1_body(z, s1, s2, g, bt, ws, wn, ps_ref, pn_ref):
    mu = s1[...] * (1.0 / N)
    var = s2[...] * (1.0 / N) - mu * mu
    sc = g[...] * lax.rsqrt(var + 1e-5)
    h1 = jnp.maximum((z[...] - mu) * sc + bt[...], 0.0)
    h1b = h1.astype(jnp.bfloat16)
    ps_ref[...] = jnp.dot(h1b, ws[...], preferred_element_type=jnp.float32)
    pn = jnp.dot(h1b, wn[...], preferred_element_type=jnp.float32)
    pn_ref[0] = pn[:, :HALF]
    pn_ref[1] = pn[:, HALF:]


def _z2_body(ps, sa, sb, dga, dgb, b, z_ref, s1_ref, s2_ref):
    i = pl.program_id(0)
    summ = jnp.concatenate([sa[...], sb[...]], axis=1)
    rdeg = 1.0 / jnp.maximum(dga[...] + dgb[...], 1.0)
    z = ps[...] + summ * rdeg + b[...]
    z_ref[...] = z
    bs1 = jnp.sum(z, axis=0, keepdims=True)
    bs2 = jnp.sum(z * z, axis=0, keepdims=True)

    @pl.when(i == 0)
    def _():
        s1_ref[...] = bs1
        s2_ref[...] = bs2

    @pl.when(i != 0)
    def _():
        s1_ref[...] += bs1
        s2_ref[...] += bs2


def _out_body(z, t1, t2, g, bt, h0, wra, wrb, br, wc1, bc1, wc2, bc2, o_ref):
    mu = t1[...] * (1.0 / N)
    var = t2[...] * (1.0 / N) - mu * mu
    h2 = jnp.maximum((z[...] - mu) * (g[...] * lax.rsqrt(var + 1e-5)) + bt[...],
                     0.0)
    bf = jnp.bfloat16
    hf = jnp.dot(h0[...].astype(bf), wra[...], preferred_element_type=jnp.float32)
    hf += jnp.dot(h2.astype(bf), wrb[...], preferred_element_type=jnp.float32)
    hf = jnp.maximum(hf + br[...], 0.0)
    hc = jnp.maximum(jnp.dot(hf.astype(bf), wc1[...],
                             preferred_element_type=jnp.float32)
                     + bc1[...], 0.0)
    o_ref[...] = jnp.dot(hc.astype(bf), wc2[...],
                         preferred_element_type=jnp.float32) + bc2[...]


def _tc_call(body, in_specs, out_specs, out_shape):
    return pl.pallas_call(
        body,
        grid=(_GRID,),
        in_specs=in_specs,
        out_specs=out_specs,
        out_shape=out_shape,
        compiler_params=pltpu.CompilerParams(
            dimension_semantics=("arbitrary",)),
    )


def kernel(structural_features, multimodal_features, edge_index, W_in, b_in,
           W_self0, W_neigh0, b0, gamma0, beta0,
           W_self1, W_neigh1, b1, gamma1, beta1,
           W_rel, b_rel, W_c1, b_c1, W_c2, b_c2):
    f32 = jnp.float32
    src = edge_index[0]
    dst = edge_index[1]
    src_p = jnp.concatenate([src, jnp.zeros((E_PAD - E,), jnp.int32)])
    dst_p = jnp.concatenate([dst, jnp.full((E_PAD - E,), N, jnp.int32)])
    src2 = src_p.reshape(NROWS_A, CH_A)
    dst2 = dst_p.reshape(NROWS_A, CH_A)
    dst2d = dst_p.reshape(NROWS, CHUNK)
    ones_h = jnp.ones((CHUNK, HALF), f32)
    z128 = jnp.zeros((NP, HALF), f32)

    bf = jnp.bfloat16
    WsT = W_in[:, :H].T.astype(bf)
    WmT = W_in[:, H:].T.astype(bf)
    Wself0T = W_self0.T.astype(bf)
    Wneigh0T = W_neigh0.T.astype(bf)
    Wself1T = W_self1.T.astype(bf)
    Wneigh1T = W_neigh1.T.astype(bf)
    WrAT = W_rel[:, :H].T.astype(bf)
    WrBT = W_rel[:, H:].T.astype(bf)
    Wc1T = W_c1.T.astype(bf)
    Wc2T = W_c2.T.astype(bf)
    b_in2 = b_in.reshape(1, H)
    b02 = b0.reshape(1, 2 * H)
    g02 = gamma0.reshape(1, 2 * H)
    be02 = beta0.reshape(1, 2 * H)
    b12 = b1.reshape(1, H)
    g12 = gamma1.reshape(1, H)
    be12 = beta1.reshape(1, H)
    brel2 = b_rel.reshape(1, H)
    bc12 = b_c1.reshape(1, H // 2)
    bc22 = b_c2.reshape(1, 64)

    h0, h0s = _tc_call(
        _enc_body,
        in_specs=[_rows(H), _rows(H), _full((H, H)), _full((H, H)),
                  _full((1, H))],
        out_specs=[_rows(H), pl.BlockSpec((2, _BR, HALF), lambda i: (0, i, 0))],
        out_shape=[jax.ShapeDtypeStruct((N, H), f32),
                   jax.ShapeDtypeStruct((2, N, HALF), f32)],
    )(structural_features, multimodal_features, WsT, WmT, b_in2)

    (degp,) = _make_deg()(dst2d, ones_h, z128)
    dga = degp[:N, :1]
    dgb = degp[NP:NP + N, :1]

    (summ0,) = _make_agg()(h0s.reshape(2 * N, HALF), src2, dst2, z128)
    suma, sumb = summ0[:NP], summ0[NP:]

    Z1, s1, s2 = _tc_call(
        _z1_body,
        in_specs=[_rows(H), _rows(HALF), _rows(HALF), _rows(1), _rows(1),
                  _full((H, 2 * H)), _full((H, 2 * H)), _full((1, 2 * H))],
        out_specs=[_rows(2 * H), _full((1, 2 * H)), _full((1, 2 * H))],
        out_shape=[jax.ShapeDtypeStruct((N, 2 * H), f32),
                   jax.ShapeDtypeStruct((1, 2 * H), f32),
                   jax.ShapeDtypeStruct((1, 2 * H), f32)],
    )(h0, suma[:N], sumb[:N], dga, dgb, Wself0T, Wneigh0T, b02)

    P1self, p1ns = _tc_call(
        _h1_body,
        in_specs=[_rows(2 * H), _full((1, 2 * H)), _full((1, 2 * H)),
                  _full((1, 2 * H)), _full((1, 2 * H)),
                  _full((2 * H, H)), _full((2 * H, H))],
        out_specs=[_rows(H), pl.BlockSpec((2, _BR, HALF), lambda i: (0, i, 0))],
        out_shape=[jax.ShapeDtypeStruct((N, H), f32),
                   jax.ShapeDtypeStruct((2, N, HALF), f32)],
    )(Z1, s1, s2, g02, be02, Wself1T, Wneigh1T)

    (summ1,) = _make_agg()(p1ns.reshape(2 * N, HALF), src2, dst2, z128)
    s1a, s1b = summ1[:NP], summ1[NP:]

    Z2, t1, t2 = _tc_call(
        _z2_body,
        in_specs=[_rows(H), _rows(HALF), _rows(HALF), _rows(1), _rows(1),
                  _full((1, H))],
        out_specs=[_rows(H), _full((1, H)), _full((1, H))],
        out_shape=[jax.ShapeDtypeStruct((N, H), f32),
                   jax.ShapeDtypeStruct((1, H), f32),
                   jax.ShapeDtypeStruct((1, H), f32)],
    )(P1self, s1a[:N], s1b[:N], dga, dgb, b12)

    out = _tc_call(
        _out_body,
        in_specs=[_rows(H), _full((1, H)), _full((1, H)), _full((1, H)),
                  _full((1, H)), _rows(H), _full((H, H)), _full((H, H)),
                  _full((1, H)), _full((H, H // 2)), _full((1, H // 2)),
                  _full((H // 2, 64)), _full((1, 64))],
        out_specs=_rows(64),
        out_shape=jax.ShapeDtypeStruct((N, 64), f32),
    )(Z2, t1, t2, g12, be12, h0, WrAT, WrBT, brel2, Wc1T, bc12, Wc2T, bc22)

    return out


# speedup vs baseline: 1.1054x; 1.0199x over previous
import jax
import jax.numpy as jnp
from jax import lax
from jax.experimental import pallas as pl
from jax.experimental.pallas import tpu as pltpu
from jax.experimental.pallas import tpu_sc as plsc

N = 10000
E = 160000
H = 256
HALF = 128

CHUNK = 128
CH_A = 64
NS = 16
NC = 2
E_PAD = 163840
NROWS = E_PAD // CHUNK
NROWS_A = E_PAD // CH_A
CPS_A = NROWS_A // NS
NP = 10240
ROWS_PER_TILE = NP // NS

_BR = 1000
_GRID = N // _BR


def _mesh():
    return plsc.VectorSubcoreMesh(core_axis_name="c", subcore_axis_name="s")


def _make_agg():
    CPH = 40
    NPH = CPS_A // CPH

    def body(hcat, src2, dst2, z128, out, acc, idxs, idxd,
             rows0, rows1, rows2, rows3, gs0, gs1, gs2, gs3,
             ss0, ss1, ss2, ss3):
        c = lax.axis_index("c")
        s = lax.axis_index("s")
        r0 = s * ROWS_PER_TILE
        src_bias = c * N
        rows = (rows0, rows1, rows2, rows3)
        gsem = (gs0, gs1, gs2, gs3)
        ssem = (ss0, ss1, ss2, ss3)

        pltpu.sync_copy(z128.at[pl.ds(r0, ROWS_PER_TILE)],
                        acc.at[pl.ds(r0, ROWS_PER_TILE)])
        plsc.subcore_barrier()

        def gstart(j, b):
            pltpu.async_copy(hcat.at[idxs.at[j]], rows[b], gsem[b])

        def gwait(b):
            pltpu.make_async_copy(hcat.at[idxs.at[0]], rows[b],
                                  gsem[b]).wait()

        def sstart(j, b):
            pltpu.async_copy(rows[b], acc.at[idxd.at[j]], ssem[b], add=True)

        def swait(b):
            pltpu.make_async_copy(rows[b], acc.at[idxd.at[0]],
                                  ssem[b]).wait()

        def phase(ph, carry):
            base = s * CPS_A + ph * CPH
            pltpu.sync_copy(src2.at[pl.ds(base, CPH)], idxs)
            pltpu.sync_copy(dst2.at[pl.ds(base, CPH)], idxd)

            def bias_step(j, cc):
                for k in range(CH_A // 16):
                    sl = pl.ds(k * 16, 16)
                    idxs[j, sl] = idxs[j, sl] + src_bias
                return cc

            lax.fori_loop(0, CPH, bias_step, 0)

            gstart(0, 0)
            gstart(1, 1)
            gstart(2, 2)
            gwait(0)
            sstart(0, 0)
            gstart(3, 3)

            def step4(j2, cc):
                for t in range(4):
                    j = 1 + 4 * j2 + t
                    b = (1 + t) % 4
                    bn = (b + 3) % 4
                    gwait(b)
                    swait(bn)
                    gstart(j + 3, bn)
                    sstart(j, b)
                return cc

            lax.fori_loop(0, (CPH - 4) // 4, step4, 0)
            for j, b in ((CPH - 3, 1), (CPH - 2, 2), (CPH - 1, 3)):
                gwait(b)
                sstart(j, b)
            for b in range(4):
                swait(b)
            return carry

        lax.fori_loop(0, NPH, phase, 0)
        plsc.subcore_barrier()

        o0 = c * NP + r0
        pltpu.sync_copy(acc.at[pl.ds(r0, ROWS_PER_TILE)],
                        out.at[pl.ds(o0, ROWS_PER_TILE)])

    return pl.kernel(
        body,
        out_type=[jax.ShapeDtypeStruct((2 * NP, HALF), jnp.float32)],
        mesh=_mesh(),
        scratch_types=[pltpu.VMEM_SHARED((NP, HALF), jnp.float32),
                       pltpu.VMEM((CPH, CH_A), jnp.int32),
                       pltpu.VMEM((CPH, CH_A), jnp.int32)]
        + [pltpu.VMEM((CH_A, HALF), jnp.float32)] * 4
        + [pltpu.SemaphoreType.DMA] * 8)


def _make_deg():
    half_rows = NROWS // 2
    cps = half_rows // NS

    def body(dst2, ones_h, z128, out, acc, idxd, ones_v, sem):
        c = lax.axis_index("c")
        s = lax.axis_index("s")
        r0 = s * ROWS_PER_TILE
        pltpu.sync_copy(z128.at[pl.ds(r0, ROWS_PER_TILE)],
                        acc.at[pl.ds(r0, ROWS_PER_TILE)])
        pltpu.sync_copy(ones_h, ones_v)
        base = c * half_rows + s * cps
        pltpu.sync_copy(dst2.at[pl.ds(base, cps)], idxd)
        plsc.subcore_barrier()

        def start(j, carry):
            pltpu.async_copy(ones_v, acc.at[idxd.at[j]], sem, add=True)
            return carry

        lax.fori_loop(0, cps, start, 0)

        def drain(j, carry):
            pltpu.make_async_copy(ones_v, acc.at[idxd.at[0]], sem).wait()
            return carry

        lax.fori_loop(0, cps, drain, 0)
        plsc.subcore_barrier()

        o0 = c * NP + r0
        pltpu.sync_copy(acc.at[pl.ds(r0, ROWS_PER_TILE)],
                        out.at[pl.ds(o0, ROWS_PER_TILE)])

    return pl.kernel(
        body,
        out_type=[jax.ShapeDtypeStruct((2 * NP, HALF), jnp.float32)],
        mesh=_mesh(),
        scratch_types=[pltpu.VMEM_SHARED((NP, HALF), jnp.float32),
                       pltpu.VMEM((cps, CHUNK), jnp.int32),
                       pltpu.VMEM((CHUNK, HALF), jnp.float32),
                       pltpu.SemaphoreType.DMA])


def _full(shape):
    return pl.BlockSpec(shape, lambda i: (0, 0))


def _rows(width):
    return pl.BlockSpec((_BR, width), lambda i: (i, 0))


def _enc_body(s_ref, m_ref, ws, wm, b, h_ref, hs_ref):
    bf = jnp.bfloat16
    h = jnp.dot(s_ref[...].astype(bf), ws[...], preferred_element_type=jnp.float32)
    h += jnp.dot(m_ref[...].astype(bf), wm[...], preferred_element_type=jnp.float32)
    h = jnp.maximum(h + b[...], 0.0)
    h_ref[...] = h
    hs_ref[0] = h[:, :HALF]
    hs_ref[1] = h[:, HALF:]


def _z1_body(h0, sa, sb, dga, dgb, ws, wn, b, z_ref, s1_ref, s2_ref):
    i = pl.program_id(0)
    summ = jnp.concatenate([sa[...], sb[...]], axis=1)
    rdeg = 1.0 / jnp.maximum(dga[...] + dgb[...], 1.0)
    hn = summ * rdeg
    bf = jnp.bfloat16
    z = jnp.dot(h0[...].astype(bf), ws[...], preferred_element_type=jnp.float32)
    z += jnp.dot(hn.astype(bf), wn[...], preferred_element_type=jnp.float32)
    z += b[...]
    z_ref[...] = z
    bs1 = jnp.sum(z, axis=0, keepdims=True)
    bs2 = jnp.sum(z * z, axis=0, keepdims=True)

    @pl.when(i == 0)
    def _():
        s1_ref[...] = bs1
        s2_ref[...] = bs2

    @pl.when(i != 0)
    def _():
        s1_ref[...] += bs1
        s2_ref[...] += bs2


def _h1_body(z, s1, s2, g, bt, ws, wn, ps_ref, pn_ref):
    mu = s1[...] * (1.0 / N)
    var = s2[...] * (1.0 / N) - mu * mu
    sc = g[...] * lax.rsqrt(var + 1e-5)
    h1 = jnp.maximum((z[...] - mu) * sc + bt[...], 0.0)
    h1b = h1.astype(jnp.bfloat16)
    ps_ref[...] = jnp.dot(h1b, ws[...], preferred_element_type=jnp.float32)
    pn = jnp.dot(h1b, wn[...], preferred_element_type=jnp.float32)
    pn_ref[0] = pn[:, :HALF]
    pn_ref[1] = pn[:, HALF:]


def _z2_body(ps, sa, sb, dga, dgb, b, z_ref, s1_ref, s2_ref):
    i = pl.program_id(0)
    summ = jnp.concatenate([sa[...], sb[...]], axis=1)
    rdeg = 1.0 / jnp.maximum(dga[...] + dgb[...], 1.0)
    z = ps[...] + summ * rdeg + b[...]
    z_ref[...] = z
    bs1 = jnp.sum(z, axis=0, keepdims=True)
    bs2 = jnp.sum(z * z, axis=0, keepdims=True)

    @pl.when(i == 0)
    def _():
        s1_ref[...] = bs1
        s2_ref[...] = bs2

    @pl.when(i != 0)
    def _():
        s1_ref[...] += bs1
        s2_ref[...] += bs2


def _out_body(z, t1, t2, g, bt, h0, wra, wrb, br, wc1, bc1, wc2, bc2, o_ref):
    mu = t1[...] * (1.0 / N)
    var = t2[...] * (1.0 / N) - mu * mu
    h2 = jnp.maximum((z[...] - mu) * (g[...] * lax.rsqrt(var + 1e-5)) + bt[...],
                     0.0)
    bf = jnp.bfloat16
    hf = jnp.dot(h0[...].astype(bf), wra[...], preferred_element_type=jnp.float32)
    hf += jnp.dot(h2.astype(bf), wrb[...], preferred_element_type=jnp.float32)
    hf = jnp.maximum(hf + br[...], 0.0)
    hc = jnp.maximum(jnp.dot(hf.astype(bf), wc1[...],
                             preferred_element_type=jnp.float32)
                     + bc1[...], 0.0)
    o_ref[...] = jnp.dot(hc.astype(bf), wc2[...],
                         preferred_element_type=jnp.float32) + bc2[...]


def _tc_call(body, in_specs, out_specs, out_shape):
    return pl.pallas_call(
        body,
        grid=(_GRID,),
        in_specs=in_specs,
        out_specs=out_specs,
        out_shape=out_shape,
        compiler_params=pltpu.CompilerParams(
            dimension_semantics=("arbitrary",)),
    )


def kernel(structural_features, multimodal_features, edge_index, W_in, b_in,
           W_self0, W_neigh0, b0, gamma0, beta0,
           W_self1, W_neigh1, b1, gamma1, beta1,
           W_rel, b_rel, W_c1, b_c1, W_c2, b_c2):
    f32 = jnp.float32
    src = edge_index[0]
    dst = edge_index[1]
    src_p = jnp.concatenate([src, jnp.zeros((E_PAD - E,), jnp.int32)])
    dst_p = jnp.concatenate([dst, jnp.full((E_PAD - E,), N, jnp.int32)])
    src2 = src_p.reshape(NROWS_A, CH_A)
    dst2 = dst_p.reshape(NROWS_A, CH_A)
    dst2d = dst_p.reshape(NROWS, CHUNK)
    ones_h = jnp.ones((CHUNK, HALF), f32)
    z128 = jnp.zeros((NP, HALF), f32)

    bf = jnp.bfloat16
    WsT = W_in[:, :H].T.astype(bf)
    WmT = W_in[:, H:].T.astype(bf)
    Wself0T = W_self0.T.astype(bf)
    Wneigh0T = W_neigh0.T.astype(bf)
    Wself1T = W_self1.T.astype(bf)
    Wneigh1T = W_neigh1.T.astype(bf)
    WrAT = W_rel[:, :H].T.astype(bf)
    WrBT = W_rel[:, H:].T.astype(bf)
    Wc1T = W_c1.T.astype(bf)
    Wc2T = W_c2.T.astype(bf)
    b_in2 = b_in.reshape(1, H)
    b02 = b0.reshape(1, 2 * H)
    g02 = gamma0.reshape(1, 2 * H)
    be02 = beta0.reshape(1, 2 * H)
    b12 = b1.reshape(1, H)
    g12 = gamma1.reshape(1, H)
    be12 = beta1.reshape(1, H)
    brel2 = b_rel.reshape(1, H)
    bc12 = b_c1.reshape(1, H // 2)
    bc22 = b_c2.reshape(1, 64)

    h0, h0s = _tc_call(
        _enc_body,
        in_specs=[_rows(H), _rows(H), _full((H, H)), _full((H, H)),
                  _full((1, H))],
        out_specs=[_rows(H), pl.BlockSpec((2, _BR, HALF), lambda i: (0, i, 0))],
        out_shape=[jax.ShapeDtypeStruct((N, H), f32),
                   jax.ShapeDtypeStruct((2, N, HALF), f32)],
    )(structural_features, multimodal_features, WsT, WmT, b_in2)

    (degp,) = _make_deg()(dst2d, ones_h, z128)
    dga = degp[:N, :1]
    dgb = degp[NP:NP + N, :1]

    (summ0,) = _make_agg()(h0s.reshape(2 * N, HALF), src2, dst2, z128)
    suma, sumb = summ0[:NP], summ0[NP:]

    Z1, s1, s2 = _tc_call(
        _z1_body,
        in_specs=[_rows(H), _rows(HALF), _rows(HALF), _rows(1), _rows(1),
                  _full((H, 2 * H)), _full((H, 2 * H)), _full((1, 2 * H))],
        out_specs=[_rows(2 * H), _full((1, 2 * H)), _full((1, 2 * H))],
        out_shape=[jax.ShapeDtypeStruct((N, 2 * H), f32),
                   jax.ShapeDtypeStruct((1, 2 * H), f32),
                   jax.ShapeDtypeStruct((1, 2 * H), f32)],
    )(h0, suma[:N], sumb[:N], dga, dgb, Wself0T, Wneigh0T, b02)

    P1self, p1ns = _tc_call(
        _h1_body,
        in_specs=[_rows(2 * H), _full((1, 2 * H)), _full((1, 2 * H)),
                  _full((1, 2 * H)), _full((1, 2 * H)),
                  _full((2 * H, H)), _full((2 * H, H))],
        out_specs=[_rows(H), pl.BlockSpec((2, _BR, HALF), lambda i: (0, i, 0))],
        out_shape=[jax.ShapeDtypeStruct((N, H), f32),
                   jax.ShapeDtypeStruct((2, N, HALF), f32)],
    )(Z1, s1, s2, g02, be02, Wself1T, Wneigh1T)

    (summ1,) = _make_agg()(p1ns.reshape(2 * N, HALF), src2, dst2, z128)
    s1a, s1b = summ1[:NP], summ1[NP:]

    Z2, t1, t2 = _tc_call(
        _z2_body,
        in_specs=[_rows(H), _rows(HALF), _rows(HALF), _rows(1), _rows(1),
                  _full((1, H))],
        out_specs=[_rows(H), _full((1, H)), _full((1, H))],
        out_shape=[jax.ShapeDtypeStruct((N, H), f32),
                   jax.ShapeDtypeStruct((1, H), f32),
                   jax.ShapeDtypeStruct((1, H), f32)],
    )(P1self, s1a[:N], s1b[:N], dga, dgb, b12)

    out = _tc_call(
        _out_body,
        in_specs=[_rows(H), _full((1, H)), _full((1, H)), _full((1, H)),
                  _full((1, H)), _rows(H), _full((H, H)), _full((H, H)),
                  _full((1, H)), _full((H, H // 2)), _full((1, H // 2)),
                  _full((H // 2, 64)), _full((1, 64))],
        out_specs=_rows(64),
        out_shape=jax.ShapeDtypeStruct((N, 64), f32),
    )(Z2, t1, t2, g12, be12, h0, WrAT, WrBT, brel2, Wc1T, bc12, Wc2T, bc22)

    return out


# speedup vs baseline: 1.1158x; 1.0093x over previous
import jax
import jax.numpy as jnp
from jax import lax
from jax.experimental import pallas as pl
from jax.experimental.pallas import tpu as pltpu
from jax.experimental.pallas import tpu_sc as plsc

N = 10000
E = 160000
H = 256
HALF = 128

CHUNK = 128
CH_A = 64
NS = 16
NC = 2
E_PAD = 163840
NROWS = E_PAD // CHUNK
NROWS_A = E_PAD // CH_A
CPS_A = NROWS_A // NS
NP = 10240
ROWS_PER_TILE = NP // NS

_BR = 2000
_GRID = N // _BR


def _mesh():
    return plsc.VectorSubcoreMesh(core_axis_name="c", subcore_axis_name="s")


def _make_agg():
    CPH = 40
    NPH = CPS_A // CPH

    def body(hcat, src2, dst2, z128, out, acc, idxs, idxd,
             rows0, rows1, rows2, rows3, gs0, gs1, gs2, gs3,
             ss0, ss1, ss2, ss3):
        c = lax.axis_index("c")
        s = lax.axis_index("s")
        r0 = s * ROWS_PER_TILE
        src_bias = c * N
        rows = (rows0, rows1, rows2, rows3)
        gsem = (gs0, gs1, gs2, gs3)
        ssem = (ss0, ss1, ss2, ss3)

        pltpu.sync_copy(z128.at[pl.ds(r0, ROWS_PER_TILE)],
                        acc.at[pl.ds(r0, ROWS_PER_TILE)])
        plsc.subcore_barrier()

        def gstart(j, b):
            pltpu.async_copy(hcat.at[idxs.at[j]], rows[b], gsem[b])

        def gwait(b):
            pltpu.make_async_copy(hcat.at[idxs.at[0]], rows[b],
                                  gsem[b]).wait()

        def sstart(j, b):
            pltpu.async_copy(rows[b], acc.at[idxd.at[j]], ssem[b], add=True)

        def swait(b):
            pltpu.make_async_copy(rows[b], acc.at[idxd.at[0]],
                                  ssem[b]).wait()

        def phase(ph, carry):
            base = s * CPS_A + ph * CPH
            pltpu.sync_copy(src2.at[pl.ds(base, CPH)], idxs)
            pltpu.sync_copy(dst2.at[pl.ds(base, CPH)], idxd)

            def bias_step(j, cc):
                for k in range(CH_A // 16):
                    sl = pl.ds(k * 16, 16)
                    idxs[j, sl] = idxs[j, sl] + src_bias
                return cc

            lax.fori_loop(0, CPH, bias_step, 0)

            gstart(0, 0)
            gstart(1, 1)
            gstart(2, 2)
            gwait(0)
            sstart(0, 0)
            gstart(3, 3)

            def step4(j2, cc):
                for t in range(4):
                    j = 1 + 4 * j2 + t
                    b = (1 + t) % 4
                    bn = (b + 3) % 4
                    gwait(b)
                    swait(bn)
                    gstart(j + 3, bn)
                    sstart(j, b)
                return cc

            lax.fori_loop(0, (CPH - 4) // 4, step4, 0)
            for j, b in ((CPH - 3, 1), (CPH - 2, 2), (CPH - 1, 3)):
                gwait(b)
                sstart(j, b)
            for b in range(4):
                swait(b)
            return carry

        lax.fori_loop(0, NPH, phase, 0)
        plsc.subcore_barrier()

        o0 = c * NP + r0
        pltpu.sync_copy(acc.at[pl.ds(r0, ROWS_PER_TILE)],
                        out.at[pl.ds(o0, ROWS_PER_TILE)])

    return pl.kernel(
        body,
        out_type=[jax.ShapeDtypeStruct((2 * NP, HALF), jnp.float32)],
        mesh=_mesh(),
        scratch_types=[pltpu.VMEM_SHARED((NP, HALF), jnp.float32),
                       pltpu.VMEM((CPH, CH_A), jnp.int32),
                       pltpu.VMEM((CPH, CH_A), jnp.int32)]
        + [pltpu.VMEM((CH_A, HALF), jnp.float32)] * 4
        + [pltpu.SemaphoreType.DMA] * 8)


def _make_deg():
    half_rows = NROWS // 2
    cps = half_rows // NS

    def body(dst2, ones_h, z128, out, acc, idxd, ones_v, sem):
        c = lax.axis_index("c")
        s = lax.axis_index("s")
        r0 = s * ROWS_PER_TILE
        pltpu.sync_copy(z128.at[pl.ds(r0, ROWS_PER_TILE)],
                        acc.at[pl.ds(r0, ROWS_PER_TILE)])
        pltpu.sync_copy(ones_h, ones_v)
        base = c * half_rows + s * cps
        pltpu.sync_copy(dst2.at[pl.ds(base, cps)], idxd)
        plsc.subcore_barrier()

        def start(j, carry):
            pltpu.async_copy(ones_v, acc.at[idxd.at[j]], sem, add=True)
            return carry

        lax.fori_loop(0, cps, start, 0)

        def drain(j, carry):
            pltpu.make_async_copy(ones_v, acc.at[idxd.at[0]], sem).wait()
            return carry

        lax.fori_loop(0, cps, drain, 0)
        plsc.subcore_barrier()

        o0 = c * NP + r0
        pltpu.sync_copy(acc.at[pl.ds(r0, ROWS_PER_TILE)],
                        out.at[pl.ds(o0, ROWS_PER_TILE)])

    return pl.kernel(
        body,
        out_type=[jax.ShapeDtypeStruct((2 * NP, HALF), jnp.float32)],
        mesh=_mesh(),
        scratch_types=[pltpu.VMEM_SHARED((NP, HALF), jnp.float32),
                       pltpu.VMEM((cps, CHUNK), jnp.int32),
                       pltpu.VMEM((CHUNK, HALF), jnp.float32),
                       pltpu.SemaphoreType.DMA])


def _full(shape):
    return pl.BlockSpec(shape, lambda i: (0, 0))


def _rows(width):
    return pl.BlockSpec((_BR, width), lambda i: (i, 0))


def _enc_body(s_ref, m_ref, ws, wm, b, h_ref, hs_ref):
    bf = jnp.bfloat16
    h = jnp.dot(s_ref[...].astype(bf), ws[...], preferred_element_type=jnp.float32)
    h += jnp.dot(m_ref[...].astype(bf), wm[...], preferred_element_type=jnp.float32)
    h = jnp.maximum(h + b[...], 0.0)
    h_ref[...] = h
    hs_ref[0] = h[:, :HALF]
    hs_ref[1] = h[:, HALF:]


def _z1_body(h0, sa, sb, dga, dgb, ws, wn, b, z_ref, s1_ref, s2_ref):
    i = pl.program_id(0)
    summ = jnp.concatenate([sa[...], sb[...]], axis=1)
    rdeg = 1.0 / jnp.maximum(dga[...] + dgb[...], 1.0)
    hn = summ * rdeg
    bf = jnp.bfloat16
    z = jnp.dot(h0[...].astype(bf), ws[...], preferred_element_type=jnp.float32)
    z += jnp.dot(hn.astype(bf), wn[...], preferred_element_type=jnp.float32)
    z += b[...]
    z_ref[...] = z
    bs1 = jnp.sum(z, axis=0, keepdims=True)
    bs2 = jnp.sum(z * z, axis=0, keepdims=True)

    @pl.when(i == 0)
    def _():
        s1_ref[...] = bs1
        s2_ref[...] = bs2

    @pl.when(i != 0)
    def _():
        s1_ref[...] += bs1
        s2_ref[...] += bs2


def _h1_body(z, s1, s2, g, bt, ws, wn, ps_ref, pn_ref):
    mu = s1[...] * (1.0 / N)
    var = s2[...] * (1.0 / N) - mu * mu
    sc = g[...] * lax.rsqrt(var + 1e-5)
    h1 = jnp.maximum((z[...] - mu) * sc + bt[...], 0.0)
    h1b = h1.astype(jnp.bfloat16)
    ps_ref[...] = jnp.dot(h1b, ws[...], preferred_element_type=jnp.float32)
    pn = jnp.dot(h1b, wn[...], preferred_element_type=jnp.float32)
    pn_ref[0] = pn[:, :HALF]
    pn_ref[1] = pn[:, HALF:]


def _z2_body(ps, sa, sb, dga, dgb, b, z_ref, s1_ref, s2_ref):
    i = pl.program_id(0)
    summ = jnp.concatenate([sa[...], sb[...]], axis=1)
    rdeg = 1.0 / jnp.maximum(dga[...] + dgb[...], 1.0)
    z = ps[...] + summ * rdeg + b[...]
    z_ref[...] = z
    bs1 = jnp.sum(z, axis=0, keepdims=True)
    bs2 = jnp.sum(z * z, axis=0, keepdims=True)

    @pl.when(i == 0)
    def _():
        s1_ref[...] = bs1
        s2_ref[...] = bs2

    @pl.when(i != 0)
    def _():
        s1_ref[...] += bs1
        s2_ref[...] += bs2


def _out_body(z, t1, t2, g, bt, h0, wra, wrb, br, wc1, bc1, wc2, bc2, o_ref):
    mu = t1[...] * (1.0 / N)
    var = t2[...] * (1.0 / N) - mu * mu
    h2 = jnp.maximum((z[...] - mu) * (g[...] * lax.rsqrt(var + 1e-5)) + bt[...],
                     0.0)
    bf = jnp.bfloat16
    hf = jnp.dot(h0[...].astype(bf), wra[...], preferred_element_type=jnp.float32)
    hf += jnp.dot(h2.astype(bf), wrb[...], preferred_element_type=jnp.float32)
    hf = jnp.maximum(hf + br[...], 0.0)
    hc = jnp.maximum(jnp.dot(hf.astype(bf), wc1[...],
                             preferred_element_type=jnp.float32)
                     + bc1[...], 0.0)
    o_ref[...] = jnp.dot(hc.astype(bf), wc2[...],
                         preferred_element_type=jnp.float32) + bc2[...]


def _tc_call(body, in_specs, out_specs, out_shape):
    return pl.pallas_call(
        body,
        grid=(_GRID,),
        in_specs=in_specs,
        out_specs=out_specs,
        out_shape=out_shape,
        compiler_params=pltpu.CompilerParams(
            dimension_semantics=("arbitrary",)),
    )


def kernel(structural_features, multimodal_features, edge_index, W_in, b_in,
           W_self0, W_neigh0, b0, gamma0, beta0,
           W_self1, W_neigh1, b1, gamma1, beta1,
           W_rel, b_rel, W_c1, b_c1, W_c2, b_c2):
    f32 = jnp.float32
    src = edge_index[0]
    dst = edge_index[1]
    src_p = jnp.concatenate([src, jnp.zeros((E_PAD - E,), jnp.int32)])
    dst_p = jnp.concatenate([dst, jnp.full((E_PAD - E,), N, jnp.int32)])
    src2 = src_p.reshape(NROWS_A, CH_A)
    dst2 = dst_p.reshape(NROWS_A, CH_A)
    dst2d = dst_p.reshape(NROWS, CHUNK)
    ones_h = jnp.ones((CHUNK, HALF), f32)
    z128 = jnp.zeros((NP, HALF), f32)

    bf = jnp.bfloat16
    WsT = W_in[:, :H].T.astype(bf)
    WmT = W_in[:, H:].T.astype(bf)
    Wself0T = W_self0.T.astype(bf)
    Wneigh0T = W_neigh0.T.astype(bf)
    Wself1T = W_self1.T.astype(bf)
    Wneigh1T = W_neigh1.T.astype(bf)
    WrAT = W_rel[:, :H].T.astype(bf)
    WrBT = W_rel[:, H:].T.astype(bf)
    Wc1T = W_c1.T.astype(bf)
    Wc2T = W_c2.T.astype(bf)
    b_in2 = b_in.reshape(1, H)
    b02 = b0.reshape(1, 2 * H)
    g02 = gamma0.reshape(1, 2 * H)
    be02 = beta0.reshape(1, 2 * H)
    b12 = b1.reshape(1, H)
    g12 = gamma1.reshape(1, H)
    be12 = beta1.reshape(1, H)
    brel2 = b_rel.reshape(1, H)
    bc12 = b_c1.reshape(1, H // 2)
    bc22 = b_c2.reshape(1, 64)

    h0, h0s = _tc_call(
        _enc_body,
        in_specs=[_rows(H), _rows(H), _full((H, H)), _full((H, H)),
                  _full((1, H))],
        out_specs=[_rows(H), pl.BlockSpec((2, _BR, HALF), lambda i: (0, i, 0))],
        out_shape=[jax.ShapeDtypeStruct((N, H), f32),
                   jax.ShapeDtypeStruct((2, N, HALF), f32)],
    )(structural_features, multimodal_features, WsT, WmT, b_in2)

    (degp,) = _make_deg()(dst2d, ones_h, z128)
    dga = degp[:N, :1]
    dgb = degp[NP:NP + N, :1]

    (summ0,) = _make_agg()(h0s.reshape(2 * N, HALF), src2, dst2, z128)
    suma, sumb = summ0[:NP], summ0[NP:]

    Z1, s1, s2 = _tc_call(
        _z1_body,
        in_specs=[_rows(H), _rows(HALF), _rows(HALF), _rows(1), _rows(1),
                  _full((H, 2 * H)), _full((H, 2 * H)), _full((1, 2 * H))],
        out_specs=[_rows(2 * H), _full((1, 2 * H)), _full((1, 2 * H))],
        out_shape=[jax.ShapeDtypeStruct((N, 2 * H), f32),
                   jax.ShapeDtypeStruct((1, 2 * H), f32),
                   jax.ShapeDtypeStruct((1, 2 * H), f32)],
    )(h0, suma[:N], sumb[:N], dga, dgb, Wself0T, Wneigh0T, b02)

    P1self, p1ns = _tc_call(
        _h1_body,
        in_specs=[_rows(2 * H), _full((1, 2 * H)), _full((1, 2 * H)),
                  _full((1, 2 * H)), _full((1, 2 * H)),
                  _full((2 * H, H)), _full((2 * H, H))],
        out_specs=[_rows(H), pl.BlockSpec((2, _BR, HALF), lambda i: (0, i, 0))],
        out_shape=[jax.ShapeDtypeStruct((N, H), f32),
                   jax.ShapeDtypeStruct((2, N, HALF), f32)],
    )(Z1, s1, s2, g02, be02, Wself1T, Wneigh1T)

    (summ1,) = _make_agg()(p1ns.reshape(2 * N, HALF), src2, dst2, z128)
    s1a, s1b = summ1[:NP], summ1[NP:]

    Z2, t1, t2 = _tc_call(
        _z2_body,
        in_specs=[_rows(H), _rows(HALF), _rows(HALF), _rows(1), _rows(1),
                  _full((1, H))],
        out_specs=[_rows(H), _full((1, H)), _full((1, H))],
        out_shape=[jax.ShapeDtypeStruct((N, H), f32),
                   jax.ShapeDtypeStruct((1, H), f32),
                   jax.ShapeDtypeStruct((1, H), f32)],
    )(P1self, s1a[:N], s1b[:N], dga, dgb, b12)

    out = _tc_call(
        _out_body,
        in_specs=[_rows(H), _full((1, H)), _full((1, H)), _full((1, H)),
                  _full((1, H)), _rows(H), _full((H, H)), _full((H, H)),
                  _full((1, H)), _full((H, H // 2)), _full((1, H // 2)),
                  _full((H // 2, 64)), _full((1, 64))],
        out_specs=_rows(64),
        out_shape=jax.ShapeDtypeStruct((N, 64), f32),
    )(Z2, t1, t2, g12, be12, h0, WrAT, WrBT, brel2, Wc1T, bc12, Wc2T, bc22)

    return out
